# baseline (speedup 1.0000x reference)
import functools

import jax
import jax.numpy as jnp
from jax import lax
from jax.experimental import pallas as pl
from jax.experimental.pallas import tpu as pltpu
from jax.experimental.pallas import tpu_sc as plsc

N_NODES = 10000
NUM_HEDGES = 10000
NNZ = 320000
D = 128
STAR = 64
SLOPE = 0.2
NC, NS = 2, 16
NW = NC * NS
L = 16
K = 64
CHUNK = 16
B0 = 160
B1 = 160
TOT_BLOCKS = NS * (B0 + B1)
NNZ_PAD = TOT_BLOCKS * K
R_PAD = 10240
GARBAGE = 10000
RPT = R_PAD // NS
BR = 512
GRID = R_PAD // BR


def _tc1_body(x_ref, wxT_ref, bx_ref, wvT_ref, bv_ref, att_ref,
              xi_ref, g_ref, p_ref):
    i = pl.program_id(0)
    x = x_ref[...]
    xi = jnp.dot(x, wxT_ref[...], preferred_element_type=jnp.float32)
    xi_ref[...] = xi + bx_ref[...]
    xf = jnp.dot(x, wvT_ref[...], preferred_element_type=jnp.float32)
    xf = xf + bv_ref[...]
    s = jnp.dot(xf, att_ref[...], preferred_element_type=jnp.float32)
    s = jnp.where(s > 0, s, SLOPE * s)
    p = jnp.exp(s)
    rows = i * BR + lax.broadcasted_iota(jnp.int32, (BR, 1), 0)
    mask = rows < N_NODES
    g_ref[...] = jnp.where(mask, xf * p, 0.0)
    p_ref[...] = jnp.where(mask, p, 0.0)


def _tc1(X, WxT, bx, WvT, bv, attT):
    return pl.pallas_call(
        _tc1_body,
        grid=(GRID,),
        in_specs=[
            pl.BlockSpec((BR, D), lambda i: (i, 0)),
            pl.BlockSpec((D, D), lambda i: (0, 0)),
            pl.BlockSpec((1, D), lambda i: (0, 0)),
            pl.BlockSpec((D, D), lambda i: (0, 0)),
            pl.BlockSpec((1, D), lambda i: (0, 0)),
            pl.BlockSpec((D, 1), lambda i: (0, 0)),
        ],
        out_specs=[
            pl.BlockSpec((BR, D), lambda i: (i, 0)),
            pl.BlockSpec((BR, D), lambda i: (i, 0)),
            pl.BlockSpec((BR, 1), lambda i: (i, 0)),
        ],
        out_shape=[
            jax.ShapeDtypeStruct((N_NODES, D), jnp.float32),
            jax.ShapeDtypeStruct((R_PAD, D), jnp.float32),
            jax.ShapeDtypeStruct((R_PAD, 1), jnp.float32),
        ],
    )(X, WxT, bx, WvT, bv, attT)


def _tc2_body(a0_ref, a1_ref, d_ref, s_ref, t1_ref, t2_ref, tb_ref, y_ref):
    i = pl.program_id(0)
    num = a0_ref[...] + a1_ref[...]
    den = jnp.sum(d_ref[...], axis=0)[:, None]
    yv = num / jnp.maximum(den, 1e-30)
    yv = jnp.where(yv > 0, yv, jnp.exp(yv) - 1.0)
    y = jnp.dot(yv, t1_ref[...], preferred_element_type=jnp.float32)
    y = y + jnp.dot(s_ref[...], t2_ref[...], preferred_element_type=jnp.float32)
    y = y + tb_ref[...]
    rows = i * BR + lax.broadcasted_iota(jnp.int32, (BR, 1), 0)
    y_ref[...] = jnp.where(rows < NUM_HEDGES, y, 0.0)


def _tc2(a0, a1, den, S_features, th1T, th2T, tb):
    return pl.pallas_call(
        _tc2_body,
        grid=(GRID,),
        in_specs=[
            pl.BlockSpec((BR, D), lambda i: (i, 0)),
            pl.BlockSpec((BR, D), lambda i: (i, 0)),
            pl.BlockSpec((NW, BR), lambda i: (0, i)),
            pl.BlockSpec((BR, STAR), lambda i: (i, 0)),
            pl.BlockSpec((D, D), lambda i: (0, 0)),
            pl.BlockSpec((STAR, D), lambda i: (0, 0)),
            pl.BlockSpec((1, D), lambda i: (0, 0)),
        ],
        out_specs=pl.BlockSpec((BR, D), lambda i: (i, 0)),
        out_shape=jax.ShapeDtypeStruct((R_PAD, D), jnp.float32),
    )(a0, a1, den, S_features, th1T, th2T, tb)


def _tc3_body(a0_ref, a1_ref, c_ref, xi_ref, o_ref):
    sv = a0_ref[...] + a1_ref[...]
    cnt = jnp.sum(c_ref[...], axis=0)[:, None]
    m = sv / jnp.maximum(cnt, 1.0)
    m = jnp.where(m > 0, m, jnp.exp(m) - 1.0)
    o_ref[...] = m + xi_ref[...]


def _tc3(a0, a1, cnt, xi):
    return pl.pallas_call(
        _tc3_body,
        grid=(GRID,),
        in_specs=[
            pl.BlockSpec((BR, D), lambda i: (i, 0)),
            pl.BlockSpec((BR, D), lambda i: (i, 0)),
            pl.BlockSpec((NW, BR), lambda i: (0, i)),
            pl.BlockSpec((BR, D), lambda i: (i, 0)),
        ],
        out_specs=pl.BlockSpec((BR, D), lambda i: (i, 0)),
        out_shape=jax.ShapeDtypeStruct((N_NODES, D), jnp.float32),
    )(a0, a1, cnt, xi)


NBUF = 3


def _sc_pass_body(with_ptab, src_hbm, gidx_hbm, sidx_hbm,
                  ptab_hbm, out_feat, out_den, acc, gv, sv, rows, ptab, den,
                  gsems, ssems):
    c = lax.axis_index("c")
    s = lax.axis_index("s")
    wid = c * NS + s
    start_blk = jnp.where(c == 0, s * B0, NS * B0 + s * B1)
    nchunk = jnp.where(c == 0, B0 // CHUNK, B1 // CHUNK)
    z16 = jnp.zeros((L,), jnp.float32)

    def zrow(i, carry):
        for j in range(D // L):
            rows[0][i, pl.ds(j * L, L)] = z16
        return carry

    lax.fori_loop(0, K, zrow, 0)

    def zden(i, carry):
        den[pl.ds(i * L, L)] = z16
        return carry

    lax.fori_loop(0, R_PAD // L, zden, 0)
    for r in range(RPT // K):
        pltpu.sync_copy(rows[0], acc.at[pl.ds(s * RPT + r * K, K)])
    if with_ptab:
        pltpu.sync_copy(ptab_hbm.at[wid], ptab)
    plsc.subcore_barrier()

    def scalar_path(b):
        for j in range(K // L):
            e16 = sv[b, pl.ds(j * L, L)]
            if with_ptab:
                v16 = gv[b, pl.ds(j * L, L)]
                vals = plsc.load_gather(ptab, [v16])
            else:
                vals = jnp.full((L,), 1.0, jnp.float32)
            plsc.addupdate_scatter(den, [e16], vals)

    def gather(b):
        return pltpu.async_copy(src_hbm.at[gv.at[b]], rows[b % NBUF],
                                gsems[b % NBUF])

    def scatter(b):
        return pltpu.async_copy(rows[b % NBUF], acc.at[sv.at[b]],
                                ssems[b % NBUF], add=True)

    def chunk_body(ch, carry):
        blk = start_blk + ch * CHUNK
        pltpu.sync_copy(gidx_hbm.at[pl.ds(blk, CHUNK)], gv)
        pltpu.sync_copy(sidx_hbm.at[pl.ds(blk, CHUNK)], sv)
        gd = {}
        for b in range(NBUF - 1):
            gd[b] = gather(b)
        sd = {}
        for b in range(CHUNK):
            gd[b].wait()
            sd[b] = scatter(b)
            scalar_path(b)
            if b >= 1:
                sd[b - 1].wait()
            nb = b + NBUF - 1
            if nb < CHUNK:
                gd[nb] = gather(nb)
        sd[CHUNK - 1].wait()
        return carry

    lax.fori_loop(0, nchunk, chunk_body, 0)
    plsc.subcore_barrier()
    pltpu.sync_copy(acc.at[pl.ds(s * RPT, RPT)],
                    out_feat.at[c].at[pl.ds(s * RPT, RPT)])
    pltpu.sync_copy(den, out_den.at[wid])


def _make_sc_pass(with_ptab):
    scratch = [
        pltpu.VMEM_SHARED((R_PAD, D), jnp.float32),
        pltpu.VMEM((CHUNK, K), jnp.int32),
        pltpu.VMEM((CHUNK, K), jnp.int32),
    ]
    scratch += [pltpu.VMEM((K, D), jnp.float32)] * NBUF
    if with_ptab:
        scratch += [pltpu.VMEM((R_PAD,), jnp.float32)]
    scratch += [pltpu.VMEM((R_PAD,), jnp.float32)]
    scratch += [pltpu.SemaphoreType.DMA] * (2 * NBUF)

    kw = dict(
        out_type=[
            jax.ShapeDtypeStruct((NC, R_PAD, D), jnp.float32),
            jax.ShapeDtypeStruct((NW, R_PAD), jnp.float32),
        ],
        mesh=plsc.VectorSubcoreMesh(core_axis_name="c", subcore_axis_name="s"),
        compiler_params=pltpu.CompilerParams(needs_layout_passes=False),
        scratch_types=scratch,
    )

    if with_ptab:
        @functools.partial(pl.kernel, **kw)
        def sc_pass(src, gidx, sidx, ptab_hbm, out_feat, out_den,
                    acc, gv, sv, r0, r1, r2, ptab, den,
                    g0, g1, g2, s0, s1, s2):
            _sc_pass_body(True, src, gidx, sidx, ptab_hbm,
                          out_feat, out_den, acc, gv, sv, [r0, r1, r2],
                          ptab, den, [g0, g1, g2], [s0, s1, s2])
    else:
        @functools.partial(pl.kernel, **kw)
        def sc_pass(src, gidx, sidx, out_feat, out_den,
                    acc, gv, sv, r0, r1, r2, den,
                    g0, g1, g2, s0, s1, s2):
            _sc_pass_body(False, src, gidx, sidx, None,
                          out_feat, out_den, acc, gv, sv, [r0, r1, r2],
                          None, den, [g0, g1, g2], [s0, s1, s2])

    return sc_pass


_sc_pass1 = _make_sc_pass(True)
_sc_pass2 = _make_sc_pass(False)


def kernel(X, S_features, Wx_w, Wx_b, Wv_w, Wv_b, att_w, th_w, th_b,
           hyperedge_index):
    V = hyperedge_index[0]
    E = hyperedge_index[1]
    pad = jnp.full((NNZ_PAD - NNZ,), GARBAGE, jnp.int32)
    Vp = jnp.concatenate([V, pad]).reshape(TOT_BLOCKS, K)
    Ep = jnp.concatenate([E, pad]).reshape(TOT_BLOCKS, K)

    xi, g, p = _tc1(X, Wx_w.T, Wx_b[None], Wv_w.T, Wv_b[None], att_w.T)
    p_rep = jnp.broadcast_to(p.reshape(1, R_PAD), (NW, R_PAD))
    acc1, den = _sc_pass1(g, Vp, Ep, p_rep)
    y = _tc2(acc1[0], acc1[1], den, S_features, th_w[:, :D].T, th_w[:, D:].T,
             th_b[None])
    acc2, cnt = _sc_pass2(y, Ep, Vp)
    return _tc3(acc2[0], acc2[1], cnt, xi)


# speedup vs baseline: 1.0996x; 1.0996x over previous
import functools

import jax
import jax.numpy as jnp
from jax import lax
from jax.experimental import pallas as pl
from jax.experimental.pallas import tpu as pltpu
from jax.experimental.pallas import tpu_sc as plsc

N_NODES = 10000
NUM_HEDGES = 10000
NNZ = 320000
D = 128
STAR = 64
SLOPE = 0.2
NC, NS = 2, 16
NW = NC * NS
L = 16
K = 64
CHUNK = 16
B0 = 160
B1 = 160
TOT_BLOCKS = NS * (B0 + B1)
NNZ_PAD = TOT_BLOCKS * K
R_PAD = 10240
GARBAGE = 10000
RPT = R_PAD // NS
BR = 512
GRID = R_PAD // BR


def _tc1_body(x_ref, wxT_ref, bx_ref, wvT_ref, bv_ref, att_ref,
              xi_ref, g_ref, p_ref):
    i = pl.program_id(0)
    x = x_ref[...]
    xi = jnp.dot(x, wxT_ref[...], preferred_element_type=jnp.float32)
    xi_ref[...] = xi + bx_ref[...]
    xf = jnp.dot(x, wvT_ref[...], preferred_element_type=jnp.float32)
    xf = xf + bv_ref[...]
    s = jnp.dot(xf, att_ref[...], preferred_element_type=jnp.float32)
    s = jnp.where(s > 0, s, SLOPE * s)
    p = jnp.exp(s)
    rows = i * BR + lax.broadcasted_iota(jnp.int32, (BR, 1), 0)
    mask = rows < N_NODES
    g_ref[...] = jnp.where(mask, xf * p, 0.0)
    p_ref[...] = jnp.where(mask, p, 0.0)


def _tc1(X, WxT, bx, WvT, bv, attT):
    return pl.pallas_call(
        _tc1_body,
        grid=(GRID,),
        in_specs=[
            pl.BlockSpec((BR, D), lambda i: (i, 0)),
            pl.BlockSpec((D, D), lambda i: (0, 0)),
            pl.BlockSpec((1, D), lambda i: (0, 0)),
            pl.BlockSpec((D, D), lambda i: (0, 0)),
            pl.BlockSpec((1, D), lambda i: (0, 0)),
            pl.BlockSpec((D, 1), lambda i: (0, 0)),
        ],
        out_specs=[
            pl.BlockSpec((BR, D), lambda i: (i, 0)),
            pl.BlockSpec((BR, D), lambda i: (i, 0)),
            pl.BlockSpec((BR, 1), lambda i: (i, 0)),
        ],
        out_shape=[
            jax.ShapeDtypeStruct((N_NODES, D), jnp.float32),
            jax.ShapeDtypeStruct((R_PAD, D), jnp.float32),
            jax.ShapeDtypeStruct((R_PAD, 1), jnp.float32),
        ],
    )(X, WxT, bx, WvT, bv, attT)


def _tc2_body(a0_ref, a1_ref, d_ref, s_ref, t1_ref, t2_ref, tb_ref, y_ref):
    i = pl.program_id(0)
    num = a0_ref[...] + a1_ref[...]
    den = jnp.sum(d_ref[...], axis=0)[:, None]
    yv = num / jnp.maximum(den, 1e-30)
    yv = jnp.where(yv > 0, yv, jnp.exp(yv) - 1.0)
    y = jnp.dot(yv, t1_ref[...], preferred_element_type=jnp.float32)
    y = y + jnp.dot(s_ref[...], t2_ref[...], preferred_element_type=jnp.float32)
    y = y + tb_ref[...]
    rows = i * BR + lax.broadcasted_iota(jnp.int32, (BR, 1), 0)
    y_ref[...] = jnp.where(rows < NUM_HEDGES, y, 0.0)


def _tc2(a0, a1, den, S_features, th1T, th2T, tb):
    return pl.pallas_call(
        _tc2_body,
        grid=(GRID,),
        in_specs=[
            pl.BlockSpec((BR, D), lambda i: (i, 0)),
            pl.BlockSpec((BR, D), lambda i: (i, 0)),
            pl.BlockSpec((NW, BR), lambda i: (0, i)),
            pl.BlockSpec((BR, STAR), lambda i: (i, 0)),
            pl.BlockSpec((D, D), lambda i: (0, 0)),
            pl.BlockSpec((STAR, D), lambda i: (0, 0)),
            pl.BlockSpec((1, D), lambda i: (0, 0)),
        ],
        out_specs=pl.BlockSpec((BR, D), lambda i: (i, 0)),
        out_shape=jax.ShapeDtypeStruct((R_PAD, D), jnp.float32),
    )(a0, a1, den, S_features, th1T, th2T, tb)


def _tc3_body(a0_ref, a1_ref, c_ref, xi_ref, o_ref):
    sv = a0_ref[...] + a1_ref[...]
    cnt = jnp.sum(c_ref[...], axis=0)[:, None]
    m = sv / jnp.maximum(cnt, 1.0)
    m = jnp.where(m > 0, m, jnp.exp(m) - 1.0)
    o_ref[...] = m + xi_ref[...]


def _tc3(a0, a1, cnt, xi):
    return pl.pallas_call(
        _tc3_body,
        grid=(GRID,),
        in_specs=[
            pl.BlockSpec((BR, D), lambda i: (i, 0)),
            pl.BlockSpec((BR, D), lambda i: (i, 0)),
            pl.BlockSpec((NW, BR), lambda i: (0, i)),
            pl.BlockSpec((BR, D), lambda i: (i, 0)),
        ],
        out_specs=pl.BlockSpec((BR, D), lambda i: (i, 0)),
        out_shape=jax.ShapeDtypeStruct((N_NODES, D), jnp.float32),
    )(a0, a1, cnt, xi)


NBUF = 3


def _sc_pass_body(with_ptab, src_hbm, gidx_hbm, sidx_hbm,
                  ptab_hbm, out_feat, out_den, acc, gv, sv, rows, ptab, den,
                  gsems, ssems):
    c = lax.axis_index("c")
    s = lax.axis_index("s")
    wid = c * NS + s
    start_blk = jnp.where(c == 0, s * B0, NS * B0 + s * B1)
    nchunk = jnp.where(c == 0, B0 // CHUNK, B1 // CHUNK)
    z16 = jnp.zeros((L,), jnp.float32)

    def zrow(i, carry):
        for j in range(D // L):
            rows[0][i, pl.ds(j * L, L)] = z16
        return carry

    lax.fori_loop(0, K, zrow, 0)

    def zden(i, carry):
        den[pl.ds(i * L, L)] = z16
        return carry

    lax.fori_loop(0, R_PAD // L, zden, 0)
    for r in range(RPT // K):
        pltpu.sync_copy(rows[0], acc.at[pl.ds(s * RPT + r * K, K)])
    if with_ptab:
        pltpu.sync_copy(ptab_hbm.at[wid], ptab)
    plsc.subcore_barrier()

    def scalar_path(b):
        for j in range(K // L):
            e16 = sv[b, pl.ds(j * L, L)]
            if with_ptab:
                v16 = gv[b, pl.ds(j * L, L)]
                vals = plsc.load_gather(ptab, [v16])
            else:
                vals = jnp.full((L,), 1.0, jnp.float32)
            plsc.addupdate_scatter(den, [e16], vals)

    def gather(b):
        return pltpu.async_copy(src_hbm.at[gv.at[b]], rows[b % NBUF],
                                gsems[b % NBUF])

    def scatter(b):
        return pltpu.async_copy(rows[b % NBUF], acc.at[sv.at[b]],
                                ssems[b % NBUF], add=True)

    def chunk_body(ch, carry):
        blk = (ch * NW + wid) * CHUNK
        pltpu.sync_copy(gidx_hbm.at[pl.ds(blk, CHUNK)], gv)
        pltpu.sync_copy(sidx_hbm.at[pl.ds(blk, CHUNK)], sv)
        gd = {}
        for b in range(NBUF - 1):
            gd[b] = gather(b)
        sd = {}
        for b in range(CHUNK):
            gd[b].wait()
            sd[b] = scatter(b)
            scalar_path(b)
            if b >= 1:
                sd[b - 1].wait()
            nb = b + NBUF - 1
            if nb < CHUNK:
                gd[nb] = gather(nb)
        sd[CHUNK - 1].wait()
        return carry

    lax.fori_loop(0, nchunk, chunk_body, 0)
    plsc.subcore_barrier()
    pltpu.sync_copy(acc.at[pl.ds(s * RPT, RPT)],
                    out_feat.at[c].at[pl.ds(s * RPT, RPT)])
    pltpu.sync_copy(den, out_den.at[wid])


def _make_sc_pass(with_ptab):
    scratch = [
        pltpu.VMEM_SHARED((R_PAD, D), jnp.float32),
        pltpu.VMEM((CHUNK, K), jnp.int32),
        pltpu.VMEM((CHUNK, K), jnp.int32),
    ]
    scratch += [pltpu.VMEM((K, D), jnp.float32)] * NBUF
    if with_ptab:
        scratch += [pltpu.VMEM((R_PAD,), jnp.float32)]
    scratch += [pltpu.VMEM((R_PAD,), jnp.float32)]
    scratch += [pltpu.SemaphoreType.DMA] * (2 * NBUF)

    kw = dict(
        out_type=[
            jax.ShapeDtypeStruct((NC, R_PAD, D), jnp.float32),
            jax.ShapeDtypeStruct((NW, R_PAD), jnp.float32),
        ],
        mesh=plsc.VectorSubcoreMesh(core_axis_name="c", subcore_axis_name="s"),
        compiler_params=pltpu.CompilerParams(needs_layout_passes=False),
        scratch_types=scratch,
    )

    if with_ptab:
        @functools.partial(pl.kernel, **kw)
        def sc_pass(src, gidx, sidx, ptab_hbm, out_feat, out_den,
                    acc, gv, sv, r0, r1, r2, ptab, den,
                    g0, g1, g2, s0, s1, s2):
            _sc_pass_body(True, src, gidx, sidx, ptab_hbm,
                          out_feat, out_den, acc, gv, sv, [r0, r1, r2],
                          ptab, den, [g0, g1, g2], [s0, s1, s2])
    else:
        @functools.partial(pl.kernel, **kw)
        def sc_pass(src, gidx, sidx, out_feat, out_den,
                    acc, gv, sv, r0, r1, r2, den,
                    g0, g1, g2, s0, s1, s2):
            _sc_pass_body(False, src, gidx, sidx, None,
                          out_feat, out_den, acc, gv, sv, [r0, r1, r2],
                          None, den, [g0, g1, g2], [s0, s1, s2])

    return sc_pass


_sc_pass1 = _make_sc_pass(True)
_sc_pass2 = _make_sc_pass(False)


def kernel(X, S_features, Wx_w, Wx_b, Wv_w, Wv_b, att_w, th_w, th_b,
           hyperedge_index):
    V = hyperedge_index[0]
    E = hyperedge_index[1]
    pad = jnp.full((NNZ_PAD - NNZ,), GARBAGE, jnp.int32)
    Vp = jnp.concatenate([V, pad]).reshape(TOT_BLOCKS, K)
    Ep = jnp.concatenate([E, pad]).reshape(TOT_BLOCKS, K)

    xi, g, p = _tc1(X, Wx_w.T, Wx_b[None], Wv_w.T, Wv_b[None], att_w.T)
    p_rep = jnp.broadcast_to(p.reshape(1, R_PAD), (NW, R_PAD))
    acc1, den = _sc_pass1(g, Vp, Ep, p_rep)
    y = _tc2(acc1[0], acc1[1], den, S_features, th_w[:, :D].T, th_w[:, D:].T,
             th_b[None])
    acc2, cnt = _sc_pass2(y, Ep, Vp)
    return _tc3(acc2[0], acc2[1], cnt, xi)
